# baseline (speedup 1.0000x reference)
import functools

import jax
import jax.numpy as jnp
from jax import lax
from jax.experimental import pallas as pl
from jax.experimental.pallas import tpu as pltpu
from jax.experimental.pallas import tpu_sc as plsc

_H = 32
_W = 32
_HW = _H * _W
_HALF = 384
_DIM = 2 * _HALF
_N = 32
_NW = 32
_CHUNK = _W * _DIM
_BATCH_STRIDE = _HW * _DIM


def _sc_add(xflat, row_table, col_flat):
    mesh = plsc.VectorSubcoreMesh(core_axis_name="c", subcore_axis_name="s")

    @functools.partial(
        pl.kernel,
        out_type=jax.ShapeDtypeStruct((_N * _BATCH_STRIDE,), jnp.float32),
        mesh=mesh,
        scratch_types=[
            pltpu.VMEM((_HALF,), jnp.float32),
            pltpu.VMEM((_W * _HALF,), jnp.float32),
            pltpu.VMEM((_CHUNK,), jnp.float32),
            pltpu.VMEM((_CHUNK,), jnp.float32),
        ],
    )
    def k(x_hbm, row_hbm, col_hbm, out_hbm, rowbuf, colbuf, pos_v, x_v):
        wid = lax.axis_index("s") * 2 + lax.axis_index("c")
        pltpu.sync_copy(row_hbm.at[wid], rowbuf)
        pltpu.sync_copy(col_hbm, colbuf)

        def build_row(j, _):
            def cp_row(cb, _):
                pos_v[pl.ds(j * _DIM + cb * 16, 16)] = rowbuf[pl.ds(cb * 16, 16)]
                return 0

            def cp_col(cb, _):
                pos_v[pl.ds(j * _DIM + _HALF + cb * 16, 16)] = (
                    colbuf[pl.ds(j * _HALF + cb * 16, 16)])
                return 0

            lax.fori_loop(0, _HALF // 16, cp_row, 0)
            lax.fori_loop(0, _HALF // 16, cp_col, 0)
            return 0

        lax.fori_loop(0, _W, build_row, 0)

        base = wid * _CHUNK

        def add_vec(i, _):
            x_v[pl.ds(i * 16, 16)] = (
                x_v[pl.ds(i * 16, 16)] + pos_v[pl.ds(i * 16, 16)])
            return 0

        for b in range(_N):
            off = b * _BATCH_STRIDE + base
            pltpu.sync_copy(x_hbm.at[pl.ds(off, _CHUNK)], x_v)
            lax.fori_loop(0, _CHUNK // 16, add_vec, 0)
            pltpu.sync_copy(x_v, out_hbm.at[pl.ds(off, _CHUNK)])

    return k(xflat, row_table, col_flat)


def kernel(x, row_table, col_table):
    n, c, h, w = x.shape
    xflat = jnp.transpose(x, (0, 2, 3, 1)).reshape(-1)
    out = _sc_add(xflat, row_table, col_table.reshape(-1))
    return jnp.transpose(out.reshape(n, h, w, c), (0, 3, 1, 2))


# speedup vs baseline: 8.5678x; 8.5678x over previous
import jax
import jax.numpy as jnp
from jax.experimental import pallas as pl
from jax.experimental.pallas import tpu as pltpu

_H = 32
_W = 32
_HW = _H * _W
_HALF = 384
_DIM = 2 * _HALF
_NB = 2
_K = 4


def _body(x_hbm, row_ref, col_ref, o_hbm, pos_ref, inb, outb, insem, outsem):
    f = jax.lax.broadcasted_iota(jnp.int32, (_HW, _H), 0)
    k = jax.lax.broadcasted_iota(jnp.int32, (_HW, _H), 1)
    m_row = (f // _W == k).astype(jnp.float32)
    m_col = (f % _W == k).astype(jnp.float32)
    dn = (((1,), (0,)), ((), ()))
    pos_ref[:, :_HALF] = jax.lax.dot_general(
        m_row, row_ref[...], dn, preferred_element_type=jnp.float32)
    pos_ref[:, _HALF:] = jax.lax.dot_general(
        m_col, col_ref[...], dn, preferred_element_type=jnp.float32)

    nc = x_hbm.shape[0] // _NB

    def in_copy(i):
        s = i % _K
        return pltpu.make_async_copy(
            x_hbm.at[pl.ds(i * _NB, _NB)], inb.at[s], insem.at[s])

    def out_copy(i):
        s = i % _K
        return pltpu.make_async_copy(
            outb.at[s], o_hbm.at[pl.ds(i * _NB, _NB)], outsem.at[s])

    for j in range(min(_K, nc)):
        in_copy(j).start()
    for i in range(nc):
        s = i % _K
        in_copy(i).wait()
        if i >= _K:
            out_copy(i - _K).wait()
        outb[s] = inb[s] + pos_ref[...][None]
        out_copy(i).start()
        if i + _K < nc:
            in_copy(i + _K).start()
    for i in range(max(nc - _K, 0), nc):
        out_copy(i).wait()


def kernel(x, row_table, col_table):
    n, c, h, w = x.shape
    xt = jnp.transpose(x, (0, 2, 3, 1)).reshape(n, h * w, c)
    out = pl.pallas_call(
        _body,
        in_specs=[
            pl.BlockSpec(memory_space=pl.ANY),
            pl.BlockSpec(memory_space=pltpu.VMEM),
            pl.BlockSpec(memory_space=pltpu.VMEM),
        ],
        out_specs=pl.BlockSpec(memory_space=pl.ANY),
        out_shape=jax.ShapeDtypeStruct((n, h * w, c), x.dtype),
        scratch_shapes=[
            pltpu.VMEM((h * w, c), jnp.float32),
            pltpu.VMEM((_K, _NB, h * w, c), jnp.float32),
            pltpu.VMEM((_K, _NB, h * w, c), jnp.float32),
            pltpu.SemaphoreType.DMA((_K,)),
            pltpu.SemaphoreType.DMA((_K,)),
        ],
    )(xt, row_table, col_table)
    return jnp.transpose(out.reshape(n, h, w, c), (0, 3, 1, 2))


# speedup vs baseline: 8.7743x; 1.0241x over previous
import jax
import jax.numpy as jnp
from jax.experimental import pallas as pl
from jax.experimental.pallas import tpu as pltpu

_H = 32
_W = 32
_HW = _H * _W
_HALF = 384
_DIM = 2 * _HALF


def _body(x_ref, row_ref, col_ref, o_ref, pos_ref):
    @pl.when(pl.program_id(0) == 0)
    def _init_pos():
        f = jax.lax.broadcasted_iota(jnp.int32, (_HW, _H), 0)
        k = jax.lax.broadcasted_iota(jnp.int32, (_HW, _H), 1)
        m_row = (f // _W == k).astype(jnp.float32)
        m_col = (f % _W == k).astype(jnp.float32)
        dn = (((1,), (0,)), ((), ()))
        pos_ref[:, :_HALF] = jax.lax.dot_general(
            m_row, row_ref[...], dn, preferred_element_type=jnp.float32)
        pos_ref[:, _HALF:] = jax.lax.dot_general(
            m_col, col_ref[...], dn, preferred_element_type=jnp.float32)

    o_ref[...] = x_ref[...] + pos_ref[...][None]


def kernel(x, row_table, col_table):
    n, c, h, w = x.shape
    xt = jnp.transpose(x, (0, 2, 3, 1)).reshape(n, h * w, c)
    nb = 4
    out = pl.pallas_call(
        _body,
        grid=(n // nb,),
        in_specs=[
            pl.BlockSpec((nb, h * w, c), lambda b: (b, 0, 0)),
            pl.BlockSpec((h, _HALF), lambda b: (0, 0)),
            pl.BlockSpec((w, _HALF), lambda b: (0, 0)),
        ],
        out_specs=pl.BlockSpec((nb, h * w, c), lambda b: (b, 0, 0)),
        out_shape=jax.ShapeDtypeStruct((n, h * w, c), x.dtype),
        scratch_shapes=[pltpu.VMEM((h * w, c), jnp.float32)],
    )(xt, row_table, col_table)
    return jnp.transpose(out.reshape(n, h, w, c), (0, 3, 1, 2))
